# Initial kernel scaffold; baseline (speedup 1.0000x reference)
import jax
import jax.numpy as jnp
from jax.experimental import pallas as pl


def kernel(edge_index, edge_type, entity_embedding, entity_embedding_bias, basis1, comp1, root1, bias1, basis2, comp2, root2, bias2, rel_embedding):
    raise NotImplementedError("write your pallas kernel here")


# speedup vs baseline: 5.3399x; 5.3399x over previous
import functools

import jax
import jax.numpy as jnp
from jax import lax
from jax.experimental import pallas as pl
from jax.experimental.pallas import tpu as pltpu
from jax.experimental.pallas import tpu_sc as plsc

NC = 2
NS = 16
NW = NC * NS


def _weights_body(comp_ref, basis_ref, root_ref, m_ref, *, r_used):
    comp = comp_ref[...][:r_used]
    basis = basis_ref[...]
    w = jnp.einsum("rb,bdh->drh", comp, basis)
    d = basis.shape[1]
    m_ref[...] = jnp.concatenate(
        [w.reshape(d, r_used * basis.shape[2]), root_ref[...]], axis=1)


def _build_m(comp, basis, root, r_used):
    d, h = basis.shape[1], basis.shape[2]
    return pl.pallas_call(
        functools.partial(_weights_body, r_used=r_used),
        out_shape=jax.ShapeDtypeStruct((d, r_used * h + h), jnp.float32),
    )(comp, basis, root)


def _pre_body(ee_ref, eb_ref, m_ref, z_ref, rt_ref, *, r_used, h):
    x = jnp.maximum(ee_ref[...] + eb_ref[...], 0.0)
    y = jnp.dot(x, m_ref[...], preferred_element_type=jnp.float32)
    blk = x.shape[0]
    ones = jnp.ones((blk, 16), jnp.float32)
    for r in range(r_used):
        z_ref[:, r, :h] = y[:, r * h:(r + 1) * h]
        z_ref[:, r, h:] = ones
    rt_ref[...] = y[:, r_used * h:]


def _pre_stage(ee, eb, m, r_used, blk=400):
    n, d = ee.shape
    h = m.shape[1] // (r_used + 1)
    grid = n // blk
    return pl.pallas_call(
        functools.partial(_pre_body, r_used=r_used, h=h),
        grid=(grid,),
        in_specs=[
            pl.BlockSpec((blk, d), lambda i: (i, 0)),
            pl.BlockSpec((1, d), lambda i: (0, 0)),
            pl.BlockSpec(m.shape, lambda i: (0, 0)),
        ],
        out_specs=[
            pl.BlockSpec((blk, r_used, h + 16), lambda i: (i, 0, 0)),
            pl.BlockSpec((blk, h), lambda i: (i, 0)),
        ],
        out_shape=[
            jax.ShapeDtypeStruct((n, r_used, h + 16), jnp.float32),
            jax.ShapeDtypeStruct((n, h), jnp.float32),
        ],
    )(ee, eb, m)


def _mid_body(aggp_ref, rt_ref, b_ref, m_ref, z_ref, rt2_ref, *, h):
    a = aggp_ref[0] + aggp_ref[1]
    deg = a[:, h]
    norm = 1.0 / jnp.maximum(deg, 1.0)
    x = jnp.maximum(a[:, :h] * norm[:, None] + rt_ref[...] + b_ref[...], 0.0)
    y = jnp.dot(x, m_ref[...], preferred_element_type=jnp.float32)
    k = z_ref.shape[1]
    z_ref[...] = y[:, :k]
    rt2_ref[...] = y[:, k:]


def _mid_stage(aggp, rt, b, m, r_used, blk=400):
    _, n, hw = aggp.shape
    h = hw - 16
    grid = n // blk
    zdim = r_used * h
    hd = m.shape[1] - zdim
    return pl.pallas_call(
        functools.partial(_mid_body, h=h),
        grid=(grid,),
        in_specs=[
            pl.BlockSpec((2, blk, hw), lambda i: (0, i, 0)),
            pl.BlockSpec((blk, h), lambda i: (i, 0)),
            pl.BlockSpec((1, h), lambda i: (0, 0)),
            pl.BlockSpec(m.shape, lambda i: (0, 0)),
        ],
        out_specs=[
            pl.BlockSpec((blk, zdim), lambda i: (i, 0)),
            pl.BlockSpec((blk, hd), lambda i: (i, 0)),
        ],
        out_shape=[
            jax.ShapeDtypeStruct((n, zdim), jnp.float32),
            jax.ShapeDtypeStruct((n, hd), jnp.float32),
        ],
    )(aggp, rt, b, m)


def _post_body(aggp_ref, degp_ref, rt_ref, b_ref, x_ref, *, h):
    deg = degp_ref[0, :, 0] + degp_ref[1, :, 0]
    norm = 1.0 / jnp.maximum(deg, 1.0)
    a = aggp_ref[0] + aggp_ref[1]
    x_ref[...] = a * norm[:, None] + rt_ref[...] + b_ref[...]


def _post_stage(aggp, degp, rt, b, blk=400):
    _, n, h = aggp.shape
    grid = n // blk
    return pl.pallas_call(
        functools.partial(_post_body, h=h),
        grid=(grid,),
        in_specs=[
            pl.BlockSpec((2, blk, h), lambda i: (0, i, 0)),
            pl.BlockSpec((2, blk, 16), lambda i: (0, i, 0)),
            pl.BlockSpec((blk, h), lambda i: (i, 0)),
            pl.BlockSpec((1, h), lambda i: (0, 0)),
        ],
        out_specs=pl.BlockSpec((blk, h), lambda i: (i, 0)),
        out_shape=jax.ShapeDtypeStruct((n, h), jnp.float32),
    )(aggp, degp, rt, b)


def _dec_body(hp_ref, hn_ref, pred_ref, loss_ref, auc_ref, *, e, grid):
    i = pl.program_id(0)
    p = jnp.sum(hp_ref[...], axis=1)
    q = jnp.sum(hn_ref[...], axis=1)
    pred_ref[...] = p.reshape(pred_ref.shape)
    part_loss = jnp.sum(jax.nn.softplus(-p)) + jnp.sum(jax.nn.softplus(q))
    part_auc = jnp.sum((p > q).astype(jnp.float32))

    @pl.when(i == 0)
    def _():
        loss_ref[...] = jnp.zeros((1, 1), jnp.float32)
        auc_ref[...] = jnp.zeros((1, 1), jnp.float32)

    loss_ref[...] += jnp.full((1, 1), part_loss)
    auc_ref[...] += jnp.full((1, 1), part_auc)

    @pl.when(i == grid - 1)
    def _():
        loss_ref[...] = loss_ref[...] * (0.5 / e)
        auc_ref[...] = auc_ref[...] * (1.0 / e)


def _dec_reduce_stage(hrtp, hrtn, blk_e=512):
    e, h = hrtp.shape
    grid = e // blk_e
    pred2, loss, auc = pl.pallas_call(
        functools.partial(_dec_body, e=e, grid=grid),
        grid=(grid,),
        in_specs=[
            pl.BlockSpec((blk_e, h), lambda i: (i, 0)),
            pl.BlockSpec((blk_e, h), lambda i: (i, 0)),
        ],
        out_specs=[
            pl.BlockSpec((1, 1, blk_e), lambda i: (i, 0, 0)),
            pl.BlockSpec((1, 1), lambda i: (0, 0)),
            pl.BlockSpec((1, 1), lambda i: (0, 0)),
        ],
        out_shape=[
            jax.ShapeDtypeStruct((grid, 1, blk_e), jnp.float32),
            jax.ShapeDtypeStruct((1, 1), jnp.float32),
            jax.ShapeDtypeStruct((1, 1), jnp.float32),
        ],
    )(hrtp, hrtn)
    return pred2.reshape(e), loss.reshape(()), auc.reshape(())


_MESH = plsc.VectorSubcoreMesh(
    core_axis_name="c", subcore_axis_name="s", num_cores=NC, num_subcores=NS)


def _make_edge_pass(n, e, r_used, w, c=80):
    epw = e // NW
    nchunk = epw // c
    nstrip = n // c
    kmax = -(-nstrip // NS)

    scratch = [
        pltpu.VMEM((c,), jnp.int32),
        pltpu.VMEM((c,), jnp.int32),
        pltpu.VMEM((c,), jnp.int32),
        pltpu.VMEM((c,), jnp.int32),
        pltpu.VMEM((c, w), jnp.float32),
        pltpu.VMEM_SHARED((n, w), jnp.float32),
        pltpu.SemaphoreType.DMA,
    ]

    @functools.partial(
        pl.kernel,
        out_type=[jax.ShapeDtypeStruct((NC, n, w), jnp.float32)],
        mesh=_MESH, scratch_types=scratch,
        compiler_params=pltpu.CompilerParams(use_tc_tiling_on_sc=False))
    def k(z_hbm, src_hbm, dst_hbm, et_hbm, agg_out,
          src_v, dst_v, et_v, zidx_v, rows_v, agg_sh, sem):
        cid = lax.axis_index("c")
        sid = lax.axis_index("s")
        wid = cid * NS + sid

        def fill_body(i, carry):
            for v in range(w // 16):
                rows_v[i, pl.ds(v * 16, 16)] = jnp.zeros((16,), jnp.float32)
            return carry
        lax.fori_loop(0, c, fill_body, 0)

        for kk in range(kmax):
            j = sid + kk * NS

            @pl.when(j < nstrip)
            def _():
                pltpu.sync_copy(rows_v, agg_sh.at[pl.ds(j * c, c)])
        plsc.subcore_barrier()

        base = wid * epw

        def chunk_body(g, carry):
            off = base + g * c
            pltpu.sync_copy(src_hbm.at[pl.ds(off, c)], src_v)
            pltpu.sync_copy(dst_hbm.at[pl.ds(off, c)], dst_v)
            pltpu.sync_copy(et_hbm.at[pl.ds(off, c)], et_v)

            def idx_body(j, carry2):
                sv = src_v[pl.ds(j * 16, 16)]
                ev = et_v[pl.ds(j * 16, 16)]
                zidx_v[pl.ds(j * 16, 16)] = sv * r_used + ev
                return carry2
            lax.fori_loop(0, c // 16, idx_body, 0)

            pltpu.async_copy(z_hbm.at[zidx_v], rows_v, sem).wait()
            pltpu.sync_copy(rows_v, agg_sh.at[dst_v], add=True)
            return carry
        lax.fori_loop(0, nchunk, chunk_body, 0)

        plsc.subcore_barrier()
        for kk in range(kmax):
            j = sid + kk * NS

            @pl.when(j < nstrip)
            def _():
                pltpu.sync_copy(agg_sh.at[pl.ds(j * c, c)], rows_v)
                pltpu.sync_copy(rows_v, agg_out.at[cid, pl.ds(j * c, c)])

    return k


def _make_decode(n, e, h, c=80):
    epw = e // NW
    nchunk = epw // c

    scratch = [
        pltpu.VMEM((c,), jnp.int32),
        pltpu.VMEM((c,), jnp.int32),
        pltpu.VMEM((c,), jnp.int32),
        pltpu.VMEM((c,), jnp.int32),
        pltpu.VMEM((c, h), jnp.float32),
        pltpu.VMEM((c, h), jnp.float32),
        pltpu.VMEM((c, h), jnp.float32),
        pltpu.VMEM((c, h), jnp.float32),
        pltpu.VMEM((c, h), jnp.float32),
        pltpu.VMEM((c, h), jnp.float32),
        pltpu.SemaphoreType.DMA,
    ]

    @functools.partial(
        pl.kernel,
        out_type=[jax.ShapeDtypeStruct((e, h), jnp.float32),
                  jax.ShapeDtypeStruct((e, h), jnp.float32)],
        mesh=_MESH, scratch_types=scratch,
        compiler_params=pltpu.CompilerParams(use_tc_tiling_on_sc=False))
    def k(x2_hbm, rel_hbm, head_hbm, tail_hbm, nt_hbm, et_hbm,
          hrtp_out, hrtn_out, hi_v, ti_v, ni_v, ri_v,
          hb, tb, nb, rb, pb, qb, sem):
        cid = lax.axis_index("c")
        sid = lax.axis_index("s")
        wid = cid * NS + sid
        base = wid * epw

        def chunk_body(g, carry):
            off = base + g * c
            pltpu.sync_copy(head_hbm.at[pl.ds(off, c)], hi_v)
            pltpu.sync_copy(tail_hbm.at[pl.ds(off, c)], ti_v)
            pltpu.sync_copy(nt_hbm.at[pl.ds(off, c)], ni_v)
            pltpu.sync_copy(et_hbm.at[pl.ds(off, c)], ri_v)
            pltpu.async_copy(x2_hbm.at[hi_v], hb, sem).wait()
            pltpu.async_copy(x2_hbm.at[ti_v], tb, sem).wait()
            pltpu.async_copy(x2_hbm.at[ni_v], nb, sem).wait()
            pltpu.async_copy(rel_hbm.at[ri_v], rb, sem).wait()

            def edge_body(i, carry2):
                for v in range(h // 16):
                    sl = pl.ds(v * 16, 16)
                    hr = hb[i, sl] * rb[i, sl]
                    pb[i, sl] = hr * tb[i, sl]
                    qb[i, sl] = hr * nb[i, sl]
                return carry2
            lax.fori_loop(0, c, edge_body, 0)

            pltpu.sync_copy(pb, hrtp_out.at[pl.ds(off, c)])
            pltpu.sync_copy(qb, hrtn_out.at[pl.ds(off, c)])
            return carry
        lax.fori_loop(0, nchunk, chunk_body, 0)

    return k


def kernel(edge_index, edge_type, entity_embedding, entity_embedding_bias,
           basis1, comp1, root1, bias1, basis2, comp2, root2, bias2,
           rel_embedding):
    n, d = entity_embedding.shape
    nb, _, h = basis1.shape
    e = edge_type.shape[0]
    r_used = rel_embedding.shape[0]

    src = edge_index[0]
    dst = edge_index[1]
    nt_idx = jnp.roll(dst, 1)

    m1 = _build_m(comp1, basis1, root1, r_used)
    m2 = _build_m(comp2, basis2, root2, r_used)

    z1, rt1 = _pre_stage(entity_embedding, entity_embedding_bias, m1, r_used)
    z1 = z1.reshape(n * r_used, h + 16)

    edge_l1 = _make_edge_pass(n, e, r_used, h + 16)
    (aggp1,) = edge_l1(z1, src, dst, edge_type)

    z2, rt2 = _mid_stage(aggp1, rt1, bias1.reshape(1, h), m2, r_used)
    z2 = z2.reshape(n * r_used, d)

    edge_l2 = _make_edge_pass(n, e, r_used, d)
    (aggp2,) = edge_l2(z2, src, dst, edge_type)

    degp = aggp1[:, :, h:]
    x2 = _post_stage(aggp2, degp, rt2, bias2.reshape(1, d))

    decode = _make_decode(n, e, d)
    hrtp, hrtn = decode(x2, rel_embedding, src, dst, nt_idx, edge_type)

    pred_logits, loss, auc = _dec_reduce_stage(hrtp, hrtn)
    return (pred_logits, loss, auc)


# speedup vs baseline: 6.2364x; 1.1679x over previous
import functools

import jax
import jax.numpy as jnp
from jax import lax
from jax.experimental import pallas as pl
from jax.experimental.pallas import tpu as pltpu
from jax.experimental.pallas import tpu_sc as plsc

NC = 2
NS = 16
NW = NC * NS


def _weights_body(comp_ref, basis_ref, root_ref, m_ref, *, r_used):
    comp = comp_ref[...][:r_used]
    basis = basis_ref[...]
    w = jnp.einsum("rb,bdh->drh", comp, basis)
    d = basis.shape[1]
    m_ref[...] = jnp.concatenate(
        [w.reshape(d, r_used * basis.shape[2]), root_ref[...]], axis=1)


def _build_m(comp, basis, root, r_used):
    d, h = basis.shape[1], basis.shape[2]
    return pl.pallas_call(
        functools.partial(_weights_body, r_used=r_used),
        out_shape=jax.ShapeDtypeStruct((d, r_used * h + h), jnp.float32),
    )(comp, basis, root)


def _pre_body(ee_ref, eb_ref, m_ref, z_ref, rt_ref, *, r_used, h):
    x = jnp.maximum(ee_ref[...] + eb_ref[...], 0.0)
    y = jnp.dot(x, m_ref[...], preferred_element_type=jnp.float32)
    blk = x.shape[0]
    ones = jnp.ones((blk, 16), jnp.float32)
    for r in range(r_used):
        z_ref[:, r, :h] = y[:, r * h:(r + 1) * h]
        z_ref[:, r, h:] = ones
    rt_ref[...] = y[:, r_used * h:]


def _pre_stage(ee, eb, m, r_used, blk=400):
    n, d = ee.shape
    h = m.shape[1] // (r_used + 1)
    grid = n // blk
    return pl.pallas_call(
        functools.partial(_pre_body, r_used=r_used, h=h),
        grid=(grid,),
        in_specs=[
            pl.BlockSpec((blk, d), lambda i: (i, 0)),
            pl.BlockSpec((1, d), lambda i: (0, 0)),
            pl.BlockSpec(m.shape, lambda i: (0, 0)),
        ],
        out_specs=[
            pl.BlockSpec((blk, r_used, h + 16), lambda i: (i, 0, 0)),
            pl.BlockSpec((blk, h), lambda i: (i, 0)),
        ],
        out_shape=[
            jax.ShapeDtypeStruct((n, r_used, h + 16), jnp.float32),
            jax.ShapeDtypeStruct((n, h), jnp.float32),
        ],
    )(ee, eb, m)


def _mid_body(aggp_ref, rt_ref, b_ref, m_ref, z_ref, rt2_ref, *, h):
    a = aggp_ref[0] + aggp_ref[1]
    deg = a[:, h]
    norm = 1.0 / jnp.maximum(deg, 1.0)
    x = jnp.maximum(a[:, :h] * norm[:, None] + rt_ref[...] + b_ref[...], 0.0)
    y = jnp.dot(x, m_ref[...], preferred_element_type=jnp.float32)
    k = z_ref.shape[1]
    z_ref[...] = y[:, :k]
    rt2_ref[...] = y[:, k:]


def _mid_stage(aggp, rt, b, m, r_used, blk=400):
    _, n, hw = aggp.shape
    h = hw - 16
    grid = n // blk
    zdim = r_used * h
    hd = m.shape[1] - zdim
    return pl.pallas_call(
        functools.partial(_mid_body, h=h),
        grid=(grid,),
        in_specs=[
            pl.BlockSpec((2, blk, hw), lambda i: (0, i, 0)),
            pl.BlockSpec((blk, h), lambda i: (i, 0)),
            pl.BlockSpec((1, h), lambda i: (0, 0)),
            pl.BlockSpec(m.shape, lambda i: (0, 0)),
        ],
        out_specs=[
            pl.BlockSpec((blk, zdim), lambda i: (i, 0)),
            pl.BlockSpec((blk, hd), lambda i: (i, 0)),
        ],
        out_shape=[
            jax.ShapeDtypeStruct((n, zdim), jnp.float32),
            jax.ShapeDtypeStruct((n, hd), jnp.float32),
        ],
    )(aggp, rt, b, m)


def _post_body(aggp_ref, degp_ref, rt_ref, b_ref, x_ref, *, h):
    deg = degp_ref[0, :, 0] + degp_ref[1, :, 0]
    norm = 1.0 / jnp.maximum(deg, 1.0)
    a = aggp_ref[0] + aggp_ref[1]
    x_ref[...] = a * norm[:, None] + rt_ref[...] + b_ref[...]


def _post_stage(aggp, degp, rt, b, blk=400):
    _, n, h = aggp.shape
    grid = n // blk
    return pl.pallas_call(
        functools.partial(_post_body, h=h),
        grid=(grid,),
        in_specs=[
            pl.BlockSpec((2, blk, h), lambda i: (0, i, 0)),
            pl.BlockSpec((2, blk, 16), lambda i: (0, i, 0)),
            pl.BlockSpec((blk, h), lambda i: (i, 0)),
            pl.BlockSpec((1, h), lambda i: (0, 0)),
        ],
        out_specs=pl.BlockSpec((blk, h), lambda i: (i, 0)),
        out_shape=jax.ShapeDtypeStruct((n, h), jnp.float32),
    )(aggp, degp, rt, b)


def _dec_body(hp_ref, hn_ref, pred_ref, loss_ref, auc_ref, *, e, grid):
    i = pl.program_id(0)
    p = jnp.sum(hp_ref[...], axis=1)
    q = jnp.sum(hn_ref[...], axis=1)
    pred_ref[...] = p.reshape(pred_ref.shape)
    part_loss = jnp.sum(jax.nn.softplus(-p)) + jnp.sum(jax.nn.softplus(q))
    part_auc = jnp.sum((p > q).astype(jnp.float32))

    @pl.when(i == 0)
    def _():
        loss_ref[...] = jnp.zeros((1, 1), jnp.float32)
        auc_ref[...] = jnp.zeros((1, 1), jnp.float32)

    loss_ref[...] += jnp.full((1, 1), part_loss)
    auc_ref[...] += jnp.full((1, 1), part_auc)

    @pl.when(i == grid - 1)
    def _():
        loss_ref[...] = loss_ref[...] * (0.5 / e)
        auc_ref[...] = auc_ref[...] * (1.0 / e)


def _dec_reduce_stage(hrtp, hrtn, blk_e=512):
    e, h = hrtp.shape
    grid = e // blk_e
    pred2, loss, auc = pl.pallas_call(
        functools.partial(_dec_body, e=e, grid=grid),
        grid=(grid,),
        in_specs=[
            pl.BlockSpec((blk_e, h), lambda i: (i, 0)),
            pl.BlockSpec((blk_e, h), lambda i: (i, 0)),
        ],
        out_specs=[
            pl.BlockSpec((1, 1, blk_e), lambda i: (i, 0, 0)),
            pl.BlockSpec((1, 1), lambda i: (0, 0)),
            pl.BlockSpec((1, 1), lambda i: (0, 0)),
        ],
        out_shape=[
            jax.ShapeDtypeStruct((grid, 1, blk_e), jnp.float32),
            jax.ShapeDtypeStruct((1, 1), jnp.float32),
            jax.ShapeDtypeStruct((1, 1), jnp.float32),
        ],
    )(hrtp, hrtn)
    return pred2.reshape(e), loss.reshape(()), auc.reshape(())


_MESH = plsc.VectorSubcoreMesh(
    core_axis_name="c", subcore_axis_name="s", num_cores=NC, num_subcores=NS)


def _make_edge_pass(n, e, r_used, w, c=80):
    epw = e // NW
    nchunk = epw // c
    nstrip = n // c
    kmax = -(-nstrip // NS)

    scratch = (
        [pltpu.VMEM((c,), jnp.int32)] * 8 +
        [pltpu.VMEM((c, w), jnp.float32)] * 2 +
        [pltpu.VMEM_SHARED((n, w), jnp.float32),
         pltpu.SemaphoreType.DMA,
         pltpu.SemaphoreType.DMA]
    )

    @functools.partial(
        pl.kernel,
        out_type=[jax.ShapeDtypeStruct((NC, n, w), jnp.float32)],
        mesh=_MESH, scratch_types=scratch,
        compiler_params=pltpu.CompilerParams(use_tc_tiling_on_sc=False))
    def k(z_hbm, src_hbm, dst_hbm, et_hbm, agg_out,
          src0, src1, dst0, dst1, et0, et1, zidx0, zidx1,
          rows0, rows1, agg_sh, sem0, sem1):
        src_v = [src0, src1]
        dst_v = [dst0, dst1]
        et_v = [et0, et1]
        zidx_v = [zidx0, zidx1]
        rows_v = [rows0, rows1]
        sem = [sem0, sem1]
        cid = lax.axis_index("c")
        sid = lax.axis_index("s")
        wid = cid * NS + sid

        def fill_body(i, carry):
            for v in range(w // 16):
                rows0[i, pl.ds(v * 16, 16)] = jnp.zeros((16,), jnp.float32)
            return carry
        lax.fori_loop(0, c, fill_body, 0)

        for kk in range(kmax):
            j = sid + kk * NS

            @pl.when(j < nstrip)
            def _():
                pltpu.sync_copy(rows0, agg_sh.at[pl.ds(j * c, c)])
        plsc.subcore_barrier()

        base = wid * epw

        def stage(q, b):
            off = base + q * c
            pltpu.sync_copy(src_hbm.at[pl.ds(off, c)], src_v[b])
            pltpu.sync_copy(dst_hbm.at[pl.ds(off, c)], dst_v[b])
            pltpu.sync_copy(et_hbm.at[pl.ds(off, c)], et_v[b])

            def idx_body(j, carry2):
                sv = src_v[b][pl.ds(j * 16, 16)]
                ev = et_v[b][pl.ds(j * 16, 16)]
                zidx_v[b][pl.ds(j * 16, 16)] = sv * r_used + ev
                return carry2
            lax.fori_loop(0, c // 16, idx_body, 0)
            pltpu.async_copy(z_hbm.at[zidx_v[b]], rows_v[b], sem[b])

        def consume(b):
            pltpu.make_async_copy(z_hbm.at[zidx_v[b]], rows_v[b],
                                  sem[b]).wait()
            pltpu.sync_copy(rows_v[b], agg_sh.at[dst_v[b]], add=True)

        stage(0, 0)

        def pair_body(g2, carry):
            for b in range(2):
                q = g2 * 2 + b

                @pl.when(q + 1 < nchunk)
                def _():
                    stage(q + 1, 1 - b)

                @pl.when(q < nchunk)
                def _():
                    consume(b)
            return carry
        lax.fori_loop(0, -(-nchunk // 2), pair_body, 0)

        plsc.subcore_barrier()
        for kk in range(kmax):
            j = sid + kk * NS

            @pl.when(j < nstrip)
            def _():
                pltpu.sync_copy(agg_sh.at[pl.ds(j * c, c)], rows0)
                pltpu.sync_copy(rows0, agg_out.at[cid, pl.ds(j * c, c)])

    return k


def _make_decode(n, e, h, c=80):
    epw = e // NW
    nchunk = epw // c

    scratch = (
        [pltpu.VMEM((c,), jnp.int32)] * 8 +
        [pltpu.VMEM((c, h), jnp.float32)] * 8 +
        [pltpu.VMEM((c, h), jnp.float32)] * 2 +
        [pltpu.SemaphoreType.DMA] * 3
    )

    @functools.partial(
        pl.kernel,
        out_type=[jax.ShapeDtypeStruct((e, h), jnp.float32),
                  jax.ShapeDtypeStruct((e, h), jnp.float32)],
        mesh=_MESH, scratch_types=scratch,
        compiler_params=pltpu.CompilerParams(use_tc_tiling_on_sc=False))
    def k(x2_hbm, rel_hbm, head_hbm, tail_hbm, nt_hbm, et_hbm,
          hrtp_out, hrtn_out, hi0, hi1, ti0, ti1, ni0, ni1, ri0, ri1,
          hb0, hb1, tb0, tb1, nb0, nb1, rb0, rb1, pb, qb,
          gsem0, gsem1, osem):
        hi_v = [hi0, hi1]
        ti_v = [ti0, ti1]
        ni_v = [ni0, ni1]
        ri_v = [ri0, ri1]
        hb = [hb0, hb1]
        tb = [tb0, tb1]
        nb = [nb0, nb1]
        rb = [rb0, rb1]
        gsem = [gsem0, gsem1]
        cid = lax.axis_index("c")
        sid = lax.axis_index("s")
        wid = cid * NS + sid
        base = wid * epw

        def stage(q, b):
            off = base + q * c
            pltpu.sync_copy(head_hbm.at[pl.ds(off, c)], hi_v[b])
            pltpu.sync_copy(tail_hbm.at[pl.ds(off, c)], ti_v[b])
            pltpu.sync_copy(nt_hbm.at[pl.ds(off, c)], ni_v[b])
            pltpu.sync_copy(et_hbm.at[pl.ds(off, c)], ri_v[b])
            pltpu.async_copy(x2_hbm.at[hi_v[b]], hb[b], gsem[b])
            pltpu.async_copy(x2_hbm.at[ti_v[b]], tb[b], gsem[b])
            pltpu.async_copy(x2_hbm.at[ni_v[b]], nb[b], gsem[b])
            pltpu.async_copy(rel_hbm.at[ri_v[b]], rb[b], gsem[b])

        def consume(q, b):
            off = base + q * c
            pltpu.make_async_copy(x2_hbm.at[hi_v[b]], hb[b], gsem[b]).wait()
            pltpu.make_async_copy(x2_hbm.at[ti_v[b]], tb[b], gsem[b]).wait()
            pltpu.make_async_copy(x2_hbm.at[ni_v[b]], nb[b], gsem[b]).wait()
            pltpu.make_async_copy(rel_hbm.at[ri_v[b]], rb[b], gsem[b]).wait()

            @pl.when(q > 0)
            def _():
                pltpu.make_async_copy(
                    pb, hrtp_out.at[pl.ds(off, c)], osem).wait()
                pltpu.make_async_copy(
                    qb, hrtn_out.at[pl.ds(off, c)], osem).wait()

            @plsc.parallel_loop(0, c, 1, unroll=4)
            def edge_body(i):
                for v in range(h // 16):
                    sl = pl.ds(v * 16, 16)
                    hr = hb[b][i, sl] * rb[b][i, sl]
                    pb[i, sl] = hr * tb[b][i, sl]
                    qb[i, sl] = hr * nb[b][i, sl]

            pltpu.async_copy(pb, hrtp_out.at[pl.ds(off, c)], osem)
            pltpu.async_copy(qb, hrtn_out.at[pl.ds(off, c)], osem)

        stage(0, 0)

        def pair_body(g2, carry):
            for b in range(2):
                q = g2 * 2 + b

                @pl.when(q + 1 < nchunk)
                def _():
                    stage(q + 1, 1 - b)

                @pl.when(q < nchunk)
                def _():
                    consume(q, b)
            return carry
        lax.fori_loop(0, -(-nchunk // 2), pair_body, 0)

        pltpu.make_async_copy(
            pb, hrtp_out.at[pl.ds(base, c)], osem).wait()
        pltpu.make_async_copy(
            qb, hrtn_out.at[pl.ds(base, c)], osem).wait()

    return k


def kernel(edge_index, edge_type, entity_embedding, entity_embedding_bias,
           basis1, comp1, root1, bias1, basis2, comp2, root2, bias2,
           rel_embedding):
    n, d = entity_embedding.shape
    nb, _, h = basis1.shape
    e = edge_type.shape[0]
    r_used = rel_embedding.shape[0]

    src = edge_index[0]
    dst = edge_index[1]
    nt_idx = jnp.roll(dst, 1)

    m1 = _build_m(comp1, basis1, root1, r_used)
    m2 = _build_m(comp2, basis2, root2, r_used)

    z1, rt1 = _pre_stage(entity_embedding, entity_embedding_bias, m1, r_used)
    z1 = z1.reshape(n * r_used, h + 16)

    edge_l1 = _make_edge_pass(n, e, r_used, h + 16)
    (aggp1,) = edge_l1(z1, src, dst, edge_type)

    z2, rt2 = _mid_stage(aggp1, rt1, bias1.reshape(1, h), m2, r_used)
    z2 = z2.reshape(n * r_used, d)

    edge_l2 = _make_edge_pass(n, e, r_used, d)
    (aggp2,) = edge_l2(z2, src, dst, edge_type)

    degp = aggp1[:, :, h:]
    x2 = _post_stage(aggp2, degp, rt2, bias2.reshape(1, d))

    decode = _make_decode(n, e, d)
    hrtp, hrtn = decode(x2, rel_embedding, src, dst, nt_idx, edge_type)

    pred_logits, loss, auc = _dec_reduce_stage(hrtp, hrtn)
    return (pred_logits, loss, auc)


# speedup vs baseline: 8.1034x; 1.2994x over previous
import functools

import jax
import jax.numpy as jnp
from jax import lax
from jax.experimental import pallas as pl
from jax.experimental.pallas import tpu as pltpu
from jax.experimental.pallas import tpu_sc as plsc

NC = 2
NS = 16
NW = NC * NS


def _weights_body(comp_ref, basis_ref, root_ref, m_ref, *, r_used):
    comp = comp_ref[...][:r_used]
    basis = basis_ref[...]
    w = jnp.einsum("rb,bdh->drh", comp, basis)
    d = basis.shape[1]
    m_ref[...] = jnp.concatenate(
        [w.reshape(d, r_used * basis.shape[2]), root_ref[...]], axis=1)


def _build_m(comp, basis, root, r_used):
    d, h = basis.shape[1], basis.shape[2]
    return pl.pallas_call(
        functools.partial(_weights_body, r_used=r_used),
        out_shape=jax.ShapeDtypeStruct((d, r_used * h + h), jnp.float32),
    )(comp, basis, root)


def _pre_body(ee_ref, eb_ref, m_ref, z_ref, rt_ref, *, r_used, h):
    x = jnp.maximum(ee_ref[...] + eb_ref[...], 0.0)
    y = jnp.dot(x, m_ref[...], preferred_element_type=jnp.float32)
    blk = x.shape[0]
    ones = jnp.ones((blk, 16), jnp.float32)
    for r in range(r_used):
        z_ref[r, :, :h] = y[:, r * h:(r + 1) * h]
        z_ref[r, :, h:] = ones
    rt_ref[...] = y[:, r_used * h:]


def _pre_stage(ee, eb, m, r_used, blk=400):
    n, d = ee.shape
    h = m.shape[1] // (r_used + 1)
    grid = n // blk
    return pl.pallas_call(
        functools.partial(_pre_body, r_used=r_used, h=h),
        grid=(grid,),
        in_specs=[
            pl.BlockSpec((blk, d), lambda i: (i, 0)),
            pl.BlockSpec((1, d), lambda i: (0, 0)),
            pl.BlockSpec(m.shape, lambda i: (0, 0)),
        ],
        out_specs=[
            pl.BlockSpec((r_used, blk, h + 16), lambda i: (0, i, 0)),
            pl.BlockSpec((blk, h), lambda i: (i, 0)),
        ],
        out_shape=[
            jax.ShapeDtypeStruct((r_used, n, h + 16), jnp.float32),
            jax.ShapeDtypeStruct((n, h), jnp.float32),
        ],
    )(ee, eb, m)


def _mid_body(aggp_ref, rt_ref, b_ref, m_ref, z_ref, rt2_ref, *, r_used, h):
    a = aggp_ref[0] + aggp_ref[1]
    deg = a[:, h]
    norm = 1.0 / jnp.maximum(deg, 1.0)
    x = jnp.maximum(a[:, :h] * norm[:, None] + rt_ref[...] + b_ref[...], 0.0)
    y = jnp.dot(x, m_ref[...], preferred_element_type=jnp.float32)
    d = z_ref.shape[2]
    for r in range(r_used):
        z_ref[r, :, :] = y[:, r * d:(r + 1) * d]
    rt2_ref[...] = y[:, r_used * d:]


def _mid_stage(aggp, rt, b, m, r_used, blk=400):
    _, n, hw = aggp.shape
    h = hw - 16
    grid = n // blk
    d = (m.shape[1] - h) // r_used
    return pl.pallas_call(
        functools.partial(_mid_body, r_used=r_used, h=h),
        grid=(grid,),
        in_specs=[
            pl.BlockSpec((2, blk, hw), lambda i: (0, i, 0)),
            pl.BlockSpec((blk, h), lambda i: (i, 0)),
            pl.BlockSpec((1, h), lambda i: (0, 0)),
            pl.BlockSpec(m.shape, lambda i: (0, 0)),
        ],
        out_specs=[
            pl.BlockSpec((r_used, blk, d), lambda i: (0, i, 0)),
            pl.BlockSpec((blk, d), lambda i: (i, 0)),
        ],
        out_shape=[
            jax.ShapeDtypeStruct((r_used, n, d), jnp.float32),
            jax.ShapeDtypeStruct((n, d), jnp.float32),
        ],
    )(aggp, rt, b, m)


def _post_body(aggp_ref, degp_ref, rt_ref, b_ref, x_ref, *, h):
    deg = degp_ref[0, :, 0] + degp_ref[1, :, 0]
    norm = 1.0 / jnp.maximum(deg, 1.0)
    a = aggp_ref[0] + aggp_ref[1]
    x_ref[...] = a * norm[:, None] + rt_ref[...] + b_ref[...]


def _post_stage(aggp, degp, rt, b, blk=400):
    _, n, h = aggp.shape
    grid = n // blk
    return pl.pallas_call(
        functools.partial(_post_body, h=h),
        grid=(grid,),
        in_specs=[
            pl.BlockSpec((2, blk, h), lambda i: (0, i, 0)),
            pl.BlockSpec((2, blk, 16), lambda i: (0, i, 0)),
            pl.BlockSpec((blk, h), lambda i: (i, 0)),
            pl.BlockSpec((1, h), lambda i: (0, 0)),
        ],
        out_specs=pl.BlockSpec((blk, h), lambda i: (i, 0)),
        out_shape=jax.ShapeDtypeStruct((n, h), jnp.float32),
    )(aggp, degp, rt, b)


def _dec_body(hp_ref, hn_ref, pred_ref, loss_ref, auc_ref, *, e, grid):
    i = pl.program_id(0)
    h = hp_ref.shape[1]
    ones = jnp.ones((h,), jnp.float32)
    p = jnp.dot(hp_ref[...], ones, preferred_element_type=jnp.float32)
    q = jnp.dot(hn_ref[...], ones, preferred_element_type=jnp.float32)
    pred_ref[...] = p.reshape(pred_ref.shape)
    part_loss = jnp.sum(jax.nn.softplus(-p)) + jnp.sum(jax.nn.softplus(q))
    part_auc = jnp.sum((p > q).astype(jnp.float32))

    @pl.when(i == 0)
    def _():
        loss_ref[...] = jnp.zeros((1, 1), jnp.float32)
        auc_ref[...] = jnp.zeros((1, 1), jnp.float32)

    loss_ref[...] += jnp.full((1, 1), part_loss)
    auc_ref[...] += jnp.full((1, 1), part_auc)

    @pl.when(i == grid - 1)
    def _():
        loss_ref[...] = loss_ref[...] * (0.5 / e)
        auc_ref[...] = auc_ref[...] * (1.0 / e)


def _dec_reduce_stage(hrtp, hrtn, blk_e=3200):
    e, h = hrtp.shape
    grid = e // blk_e
    pred2, loss, auc = pl.pallas_call(
        functools.partial(_dec_body, e=e, grid=grid),
        grid=(grid,),
        in_specs=[
            pl.BlockSpec((blk_e, h), lambda i: (i, 0)),
            pl.BlockSpec((blk_e, h), lambda i: (i, 0)),
        ],
        out_specs=[
            pl.BlockSpec((1, 1, blk_e), lambda i: (i, 0, 0)),
            pl.BlockSpec((1, 1), lambda i: (0, 0)),
            pl.BlockSpec((1, 1), lambda i: (0, 0)),
        ],
        out_shape=[
            jax.ShapeDtypeStruct((grid, 1, blk_e), jnp.float32),
            jax.ShapeDtypeStruct((1, 1), jnp.float32),
            jax.ShapeDtypeStruct((1, 1), jnp.float32),
        ],
    )(hrtp, hrtn)
    return pred2.reshape(e), loss.reshape(()), auc.reshape(())


_MESH = plsc.VectorSubcoreMesh(
    core_axis_name="c", subcore_axis_name="s", num_cores=NC, num_subcores=NS)

_SUP = 25


def _make_edge_pass(n, e, r_used, w, c=80):
    epw = e // NW
    nchunk = epw // c
    nsup = nchunk // _SUP
    nstrip = n // c
    kmax = -(-nstrip // NS)

    scratch = (
        [pltpu.VMEM((_SUP * c,), jnp.int32)] * 3 +
        [pltpu.VMEM((c,), jnp.int32)] * 4 +
        [pltpu.VMEM((c, w), jnp.float32)] * 2 +
        [pltpu.VMEM_SHARED((n, w), jnp.float32),
         pltpu.SemaphoreType.DMA,
         pltpu.SemaphoreType.DMA]
    )

    @functools.partial(
        pl.kernel,
        out_type=[jax.ShapeDtypeStruct((NC, n, w), jnp.float32)],
        mesh=_MESH, scratch_types=scratch,
        compiler_params=pltpu.CompilerParams(use_tc_tiling_on_sc=False))
    def k(z_hbm, src_hbm, dst_hbm, et_hbm, agg_out,
          srcb, dstb, etb, dst0, dst1, zidx0, zidx1,
          rows0, rows1, agg_sh, sem0, sem1):
        dst_v = [dst0, dst1]
        zidx_v = [zidx0, zidx1]
        rows_v = [rows0, rows1]
        sem = [sem0, sem1]
        cid = lax.axis_index("c")
        sid = lax.axis_index("s")
        wid = cid * NS + sid

        def fill_body(i, carry):
            for v in range(w // 16):
                rows0[i, pl.ds(v * 16, 16)] = jnp.zeros((16,), jnp.float32)
            return carry
        lax.fori_loop(0, c, fill_body, 0)

        for kk in range(kmax):
            j = sid + kk * NS

            @pl.when(j < nstrip)
            def _():
                pltpu.sync_copy(rows0, agg_sh.at[pl.ds(j * c, c)])
        plsc.subcore_barrier()

        base = wid * epw

        def stage(q, b):
            def idx_body(j, carry2):
                sl = pl.ds(q * c + j * 16, 16)
                sv = srcb[sl]
                ev = etb[sl]
                zidx_v[b][pl.ds(j * 16, 16)] = ev * n + sv
                dst_v[b][pl.ds(j * 16, 16)] = dstb[sl]
                return carry2
            lax.fori_loop(0, c // 16, idx_body, 0)
            pltpu.async_copy(z_hbm.at[zidx_v[b]], rows_v[b], sem[b])

        def consume(b):
            pltpu.make_async_copy(z_hbm.at[zidx_v[b]], rows_v[b],
                                  sem[b]).wait()
            pltpu.sync_copy(rows_v[b], agg_sh.at[dst_v[b]], add=True)

        for s in range(nsup):
            sup = base + s * _SUP * c
            pltpu.sync_copy(src_hbm.at[pl.ds(sup, _SUP * c)], srcb)
            pltpu.sync_copy(dst_hbm.at[pl.ds(sup, _SUP * c)], dstb)
            pltpu.sync_copy(et_hbm.at[pl.ds(sup, _SUP * c)], etb)
            stage(0, 0)

            def pair_body(g2, carry):
                for b in range(2):
                    q = g2 * 2 + b

                    @pl.when(q + 1 < _SUP)
                    def _():
                        stage(q + 1, 1 - b)

                    @pl.when(q < _SUP)
                    def _():
                        consume(b)
                return carry
            lax.fori_loop(0, -(-_SUP // 2), pair_body, 0)

        plsc.subcore_barrier()
        for kk in range(kmax):
            j = sid + kk * NS

            @pl.when(j < nstrip)
            def _():
                pltpu.sync_copy(agg_sh.at[pl.ds(j * c, c)], rows0)
                pltpu.sync_copy(rows0, agg_out.at[cid, pl.ds(j * c, c)])

    return k


def _make_decode(n, e, h, c=80):
    epw = e // NW
    nchunk = epw // c
    nsup = nchunk // _SUP

    scratch = (
        [pltpu.VMEM((_SUP * c,), jnp.int32)] * 4 +
        [pltpu.VMEM((c,), jnp.int32)] * 8 +
        [pltpu.VMEM((c, h), jnp.float32)] * 8 +
        [pltpu.VMEM((c, h), jnp.float32)] * 2 +
        [pltpu.SemaphoreType.DMA] * 3
    )

    @functools.partial(
        pl.kernel,
        out_type=[jax.ShapeDtypeStruct((e, h), jnp.float32),
                  jax.ShapeDtypeStruct((e, h), jnp.float32)],
        mesh=_MESH, scratch_types=scratch,
        compiler_params=pltpu.CompilerParams(use_tc_tiling_on_sc=False))
    def k(x2_hbm, rel_hbm, head_hbm, tail_hbm, nt_hbm, et_hbm,
          hrtp_out, hrtn_out, hib, tib, nib, rib,
          hi0, hi1, ti0, ti1, ni0, ni1, ri0, ri1,
          hb0, hb1, tb0, tb1, nb0, nb1, rb0, rb1, pb, qb,
          gsem0, gsem1, osem):
        hi_v = [hi0, hi1]
        ti_v = [ti0, ti1]
        ni_v = [ni0, ni1]
        ri_v = [ri0, ri1]
        hb = [hb0, hb1]
        tb = [tb0, tb1]
        nb = [nb0, nb1]
        rb = [rb0, rb1]
        gsem = [gsem0, gsem1]
        cid = lax.axis_index("c")
        sid = lax.axis_index("s")
        wid = cid * NS + sid
        base = wid * epw

        def stage(q, b):
            def idx_body(j, carry2):
                sl = pl.ds(q * c + j * 16, 16)
                dl = pl.ds(j * 16, 16)
                hi_v[b][dl] = hib[sl]
                ti_v[b][dl] = tib[sl]
                ni_v[b][dl] = nib[sl]
                ri_v[b][dl] = rib[sl]
                return carry2
            lax.fori_loop(0, c // 16, idx_body, 0)
            pltpu.async_copy(x2_hbm.at[hi_v[b]], hb[b], gsem[b])
            pltpu.async_copy(x2_hbm.at[ti_v[b]], tb[b], gsem[b])
            pltpu.async_copy(x2_hbm.at[ni_v[b]], nb[b], gsem[b])
            pltpu.async_copy(rel_hbm.at[ri_v[b]], rb[b], gsem[b])

        def consume(off, q, b, first):
            pltpu.make_async_copy(x2_hbm.at[hi_v[b]], hb[b], gsem[b]).wait()
            pltpu.make_async_copy(x2_hbm.at[ti_v[b]], tb[b], gsem[b]).wait()
            pltpu.make_async_copy(x2_hbm.at[ni_v[b]], nb[b], gsem[b]).wait()
            pltpu.make_async_copy(rel_hbm.at[ri_v[b]], rb[b], gsem[b]).wait()

            if first:
                @pl.when(q > 0)
                def _():
                    pltpu.make_async_copy(
                        pb, hrtp_out.at[pl.ds(off, c)], osem).wait()
                    pltpu.make_async_copy(
                        qb, hrtn_out.at[pl.ds(off, c)], osem).wait()
            else:
                pltpu.make_async_copy(
                    pb, hrtp_out.at[pl.ds(off, c)], osem).wait()
                pltpu.make_async_copy(
                    qb, hrtn_out.at[pl.ds(off, c)], osem).wait()

            @plsc.parallel_loop(0, c, 1, unroll=4)
            def edge_body(i):
                for v in range(h // 16):
                    sl = pl.ds(v * 16, 16)
                    hr = hb[b][i, sl] * rb[b][i, sl]
                    pb[i, sl] = hr * tb[b][i, sl]
                    qb[i, sl] = hr * nb[b][i, sl]

            pltpu.async_copy(pb, hrtp_out.at[pl.ds(off, c)], osem)
            pltpu.async_copy(qb, hrtn_out.at[pl.ds(off, c)], osem)

        for s in range(nsup):
            sup = base + s * _SUP * c
            pltpu.sync_copy(head_hbm.at[pl.ds(sup, _SUP * c)], hib)
            pltpu.sync_copy(tail_hbm.at[pl.ds(sup, _SUP * c)], tib)
            pltpu.sync_copy(nt_hbm.at[pl.ds(sup, _SUP * c)], nib)
            pltpu.sync_copy(et_hbm.at[pl.ds(sup, _SUP * c)], rib)
            stage(0, 0)

            def pair_body(g2, carry):
                for b in range(2):
                    q = g2 * 2 + b

                    @pl.when(q + 1 < _SUP)
                    def _():
                        stage(q + 1, 1 - b)

                    @pl.when(q < _SUP)
                    def _():
                        consume(sup + q * c, q, b, s == 0)
                return carry
            lax.fori_loop(0, -(-_SUP // 2), pair_body, 0)

        pltpu.make_async_copy(
            pb, hrtp_out.at[pl.ds(base, c)], osem).wait()
        pltpu.make_async_copy(
            qb, hrtn_out.at[pl.ds(base, c)], osem).wait()

    return k


def kernel(edge_index, edge_type, entity_embedding, entity_embedding_bias,
           basis1, comp1, root1, bias1, basis2, comp2, root2, bias2,
           rel_embedding):
    n, d = entity_embedding.shape
    nb, _, h = basis1.shape
    e = edge_type.shape[0]
    r_used = rel_embedding.shape[0]

    src = edge_index[0]
    dst = edge_index[1]
    nt_idx = jnp.roll(dst, 1)

    m1 = _build_m(comp1, basis1, root1, r_used)
    m2 = _build_m(comp2, basis2, root2, r_used)

    z1, rt1 = _pre_stage(entity_embedding, entity_embedding_bias, m1, r_used)
    z1 = z1.reshape(r_used * n, h + 16)

    edge_l1 = _make_edge_pass(n, e, r_used, h + 16)
    (aggp1,) = edge_l1(z1, src, dst, edge_type)

    z2, rt2 = _mid_stage(aggp1, rt1, bias1.reshape(1, h), m2, r_used)
    z2 = z2.reshape(r_used * n, d)

    edge_l2 = _make_edge_pass(n, e, r_used, d)
    (aggp2,) = edge_l2(z2, src, dst, edge_type)

    degp = aggp1[:, :, h:]
    x2 = _post_stage(aggp2, degp, rt2, bias2.reshape(1, d))

    decode = _make_decode(n, e, d)
    hrtp, hrtn = decode(x2, rel_embedding, src, dst, nt_idx, edge_type)

    pred_logits, loss, auc = _dec_reduce_stage(hrtp, hrtn)
    return (pred_logits, loss, auc)


# speedup vs baseline: 10.9505x; 1.3513x over previous
import functools

import jax
import jax.numpy as jnp
from jax import lax
from jax.experimental import pallas as pl
from jax.experimental.pallas import tpu as pltpu
from jax.experimental.pallas import tpu_sc as plsc

NC = 2
NS = 16
NW = NC * NS


def _weights_body(comp_ref, basis_ref, root_ref, m_ref, *, r_used):
    comp = comp_ref[...][:r_used]
    basis = basis_ref[...]
    w = jnp.einsum("rb,bdh->drh", comp, basis)
    d = basis.shape[1]
    m_ref[...] = jnp.concatenate(
        [w.reshape(d, r_used * basis.shape[2]), root_ref[...]], axis=1)


def _build_m(comp, basis, root, r_used):
    d, h = basis.shape[1], basis.shape[2]
    return pl.pallas_call(
        functools.partial(_weights_body, r_used=r_used),
        out_shape=jax.ShapeDtypeStruct((d, r_used * h + h), jnp.float32),
    )(comp, basis, root)


def _pre_body(ee_ref, eb_ref, m_ref, z_ref, rt_ref, *, r_used, h):
    x = jnp.maximum(ee_ref[...] + eb_ref[...], 0.0)
    y = jnp.dot(x, m_ref[...], preferred_element_type=jnp.float32)
    blk = x.shape[0]
    ones = jnp.ones((blk, 16), jnp.float32)
    for r in range(r_used):
        z_ref[r, :, :h] = y[:, r * h:(r + 1) * h]
        z_ref[r, :, h:] = ones
    rt_ref[...] = y[:, r_used * h:]


def _pre_stage(ee, eb, m, r_used, blk=400):
    n, d = ee.shape
    h = m.shape[1] // (r_used + 1)
    grid = n // blk
    return pl.pallas_call(
        functools.partial(_pre_body, r_used=r_used, h=h),
        grid=(grid,),
        in_specs=[
            pl.BlockSpec((blk, d), lambda i: (i, 0)),
            pl.BlockSpec((1, d), lambda i: (0, 0)),
            pl.BlockSpec(m.shape, lambda i: (0, 0)),
        ],
        out_specs=[
            pl.BlockSpec((r_used, blk, h + 16), lambda i: (0, i, 0)),
            pl.BlockSpec((blk, h), lambda i: (i, 0)),
        ],
        out_shape=[
            jax.ShapeDtypeStruct((r_used, n, h + 16), jnp.float32),
            jax.ShapeDtypeStruct((n, h), jnp.float32),
        ],
    )(ee, eb, m)


def _mid_body(aggp_ref, rt_ref, b_ref, m_ref, z_ref, rt2_ref, *, r_used, h):
    a = aggp_ref[0] + aggp_ref[1]
    deg = a[:, h]
    norm = 1.0 / jnp.maximum(deg, 1.0)
    x = jnp.maximum(a[:, :h] * norm[:, None] + rt_ref[...] + b_ref[...], 0.0)
    y = jnp.dot(x, m_ref[...], preferred_element_type=jnp.float32)
    d = z_ref.shape[2]
    for r in range(r_used):
        z_ref[r, :, :] = y[:, r * d:(r + 1) * d]
    rt2_ref[...] = y[:, r_used * d:]


def _mid_stage(aggp, rt, b, m, r_used, blk=400):
    _, n, hw = aggp.shape
    h = hw - 16
    grid = n // blk
    d = (m.shape[1] - h) // r_used
    return pl.pallas_call(
        functools.partial(_mid_body, r_used=r_used, h=h),
        grid=(grid,),
        in_specs=[
            pl.BlockSpec((2, blk, hw), lambda i: (0, i, 0)),
            pl.BlockSpec((blk, h), lambda i: (i, 0)),
            pl.BlockSpec((1, h), lambda i: (0, 0)),
            pl.BlockSpec(m.shape, lambda i: (0, 0)),
        ],
        out_specs=[
            pl.BlockSpec((r_used, blk, d), lambda i: (0, i, 0)),
            pl.BlockSpec((blk, d), lambda i: (i, 0)),
        ],
        out_shape=[
            jax.ShapeDtypeStruct((r_used, n, d), jnp.float32),
            jax.ShapeDtypeStruct((n, d), jnp.float32),
        ],
    )(aggp, rt, b, m)


def _post_body(aggp_ref, degp_ref, rt_ref, b_ref, x_ref, *, h):
    deg = degp_ref[0, :, 0] + degp_ref[1, :, 0]
    norm = 1.0 / jnp.maximum(deg, 1.0)
    a = aggp_ref[0] + aggp_ref[1]
    x_ref[...] = a * norm[:, None] + rt_ref[...] + b_ref[...]


def _post_stage(aggp, degp, rt, b, blk=400):
    _, n, h = aggp.shape
    grid = n // blk
    return pl.pallas_call(
        functools.partial(_post_body, h=h),
        grid=(grid,),
        in_specs=[
            pl.BlockSpec((2, blk, h), lambda i: (0, i, 0)),
            pl.BlockSpec((2, blk, 16), lambda i: (0, i, 0)),
            pl.BlockSpec((blk, h), lambda i: (i, 0)),
            pl.BlockSpec((1, h), lambda i: (0, 0)),
        ],
        out_specs=pl.BlockSpec((blk, h), lambda i: (i, 0)),
        out_shape=jax.ShapeDtypeStruct((n, h), jnp.float32),
    )(aggp, degp, rt, b)


def _dec_body(h_ref, t_ref, nt_ref, rel_ref, et_ref, pred_ref, loss_ref,
              auc_ref, *, e, grid):
    i = pl.program_id(0)
    r_used = rel_ref.shape[0]
    hv = h_ref[...]
    ht = hv * t_ref[...]
    hnt = hv * nt_ref[...]
    rel = rel_ref[...]
    pr = jnp.einsum("ek,rk->er", ht, rel,
                    preferred_element_type=jnp.float32)
    nr = jnp.einsum("ek,rk->er", hnt, rel,
                    preferred_element_type=jnp.float32)
    et = et_ref[0, 0, :]
    oh = (et[:, None] == lax.broadcasted_iota(jnp.int32, (1, r_used), 1)
          ).astype(jnp.float32)
    ones = jnp.ones((r_used,), jnp.float32)
    p = jnp.dot(pr * oh, ones, preferred_element_type=jnp.float32)
    q = jnp.dot(nr * oh, ones, preferred_element_type=jnp.float32)
    pred_ref[...] = p.reshape(pred_ref.shape)
    part_loss = jnp.sum(jax.nn.softplus(-p)) + jnp.sum(jax.nn.softplus(q))
    part_auc = jnp.sum((p > q).astype(jnp.float32))

    @pl.when(i == 0)
    def _():
        loss_ref[...] = jnp.zeros((1, 1), jnp.float32)
        auc_ref[...] = jnp.zeros((1, 1), jnp.float32)

    loss_ref[...] += jnp.full((1, 1), part_loss)
    auc_ref[...] += jnp.full((1, 1), part_auc)

    @pl.when(i == grid - 1)
    def _():
        loss_ref[...] = loss_ref[...] * (0.5 / e)
        auc_ref[...] = auc_ref[...] * (1.0 / e)


def _dec_reduce_stage(harr, tarr, ntarr, rel, et, blk_e=3200):
    e, h = harr.shape
    grid = e // blk_e
    et3 = et.reshape(grid, 1, blk_e)
    pred2, loss, auc = pl.pallas_call(
        functools.partial(_dec_body, e=e, grid=grid),
        grid=(grid,),
        in_specs=[
            pl.BlockSpec((blk_e, h), lambda i: (i, 0)),
            pl.BlockSpec((blk_e, h), lambda i: (i, 0)),
            pl.BlockSpec((blk_e, h), lambda i: (i, 0)),
            pl.BlockSpec(rel.shape, lambda i: (0, 0)),
            pl.BlockSpec((1, 1, blk_e), lambda i: (i, 0, 0)),
        ],
        out_specs=[
            pl.BlockSpec((1, 1, blk_e), lambda i: (i, 0, 0)),
            pl.BlockSpec((1, 1), lambda i: (0, 0)),
            pl.BlockSpec((1, 1), lambda i: (0, 0)),
        ],
        out_shape=[
            jax.ShapeDtypeStruct((grid, 1, blk_e), jnp.float32),
            jax.ShapeDtypeStruct((1, 1), jnp.float32),
            jax.ShapeDtypeStruct((1, 1), jnp.float32),
        ],
    )(harr, tarr, ntarr, rel, et3)
    return pred2.reshape(e), loss.reshape(()), auc.reshape(())


_MESH = plsc.VectorSubcoreMesh(
    core_axis_name="c", subcore_axis_name="s", num_cores=NC, num_subcores=NS)

_SUP = 25


def _make_edge_pass(n, e, r_used, w, c=80):
    epw = e // NW
    nchunk = epw // c
    nsup = nchunk // _SUP
    nstrip = n // c
    kmax = -(-nstrip // NS)

    scratch = (
        [pltpu.VMEM((_SUP * c,), jnp.int32)] * 3 +
        [pltpu.VMEM((c,), jnp.int32)] * 4 +
        [pltpu.VMEM((c, w), jnp.float32)] * 2 +
        [pltpu.VMEM_SHARED((n, w), jnp.float32),
         pltpu.SemaphoreType.DMA,
         pltpu.SemaphoreType.DMA]
    )

    @functools.partial(
        pl.kernel,
        out_type=[jax.ShapeDtypeStruct((NC, n, w), jnp.float32)],
        mesh=_MESH, scratch_types=scratch,
        compiler_params=pltpu.CompilerParams(use_tc_tiling_on_sc=False))
    def k(z_hbm, src_hbm, dst_hbm, et_hbm, agg_out,
          srcb, dstb, etb, dst0, dst1, zidx0, zidx1,
          rows0, rows1, agg_sh, sem0, sem1):
        dst_v = [dst0, dst1]
        zidx_v = [zidx0, zidx1]
        rows_v = [rows0, rows1]
        sem = [sem0, sem1]
        cid = lax.axis_index("c")
        sid = lax.axis_index("s")
        wid = cid * NS + sid

        def fill_body(i, carry):
            for v in range(w // 16):
                rows0[i, pl.ds(v * 16, 16)] = jnp.zeros((16,), jnp.float32)
            return carry
        lax.fori_loop(0, c, fill_body, 0)

        for kk in range(kmax):
            j = sid + kk * NS

            @pl.when(j < nstrip)
            def _():
                pltpu.sync_copy(rows0, agg_sh.at[pl.ds(j * c, c)])
        plsc.subcore_barrier()

        base = wid * epw

        def stage(q, b):
            def idx_body(j, carry2):
                sl = pl.ds(q * c + j * 16, 16)
                sv = srcb[sl]
                ev = etb[sl]
                zidx_v[b][pl.ds(j * 16, 16)] = ev * n + sv
                dst_v[b][pl.ds(j * 16, 16)] = dstb[sl]
                return carry2
            lax.fori_loop(0, c // 16, idx_body, 0)
            pltpu.async_copy(z_hbm.at[zidx_v[b]], rows_v[b], sem[b])

        def consume(b):
            pltpu.make_async_copy(z_hbm.at[zidx_v[b]], rows_v[b],
                                  sem[b]).wait()
            pltpu.sync_copy(rows_v[b], agg_sh.at[dst_v[b]], add=True)

        for s in range(nsup):
            sup = base + s * _SUP * c
            pltpu.sync_copy(src_hbm.at[pl.ds(sup, _SUP * c)], srcb)
            pltpu.sync_copy(dst_hbm.at[pl.ds(sup, _SUP * c)], dstb)
            pltpu.sync_copy(et_hbm.at[pl.ds(sup, _SUP * c)], etb)
            stage(0, 0)

            def pair_body(g2, carry):
                for b in range(2):
                    q = g2 * 2 + b

                    @pl.when(q + 1 < _SUP)
                    def _():
                        stage(q + 1, 1 - b)

                    @pl.when(q < _SUP)
                    def _():
                        consume(b)
                return carry
            lax.fori_loop(0, -(-_SUP // 2), pair_body, 0)

        plsc.subcore_barrier()
        for kk in range(kmax):
            j = sid + kk * NS

            @pl.when(j < nstrip)
            def _():
                pltpu.sync_copy(agg_sh.at[pl.ds(j * c, c)], rows0)
                pltpu.sync_copy(rows0, agg_out.at[cid, pl.ds(j * c, c)])

    return k


def _make_decode(n, e, h, c=80):
    epw = e // NW
    nchunk = epw // c
    nsup = nchunk // _SUP

    scratch = (
        [pltpu.VMEM((_SUP * c,), jnp.int32)] * 3 +
        [pltpu.VMEM((c,), jnp.int32)] * 6 +
        [pltpu.VMEM((c, h), jnp.float32)] * 6 +
        [pltpu.SemaphoreType.DMA] * 4
    )

    @functools.partial(
        pl.kernel,
        out_type=[jax.ShapeDtypeStruct((e, h), jnp.float32),
                  jax.ShapeDtypeStruct((e, h), jnp.float32),
                  jax.ShapeDtypeStruct((e, h), jnp.float32)],
        mesh=_MESH, scratch_types=scratch,
        compiler_params=pltpu.CompilerParams(use_tc_tiling_on_sc=False))
    def k(x2_hbm, head_hbm, tail_hbm, nt_hbm,
          h_out, t_out, nt_out, hib, tib, nib,
          hi0, hi1, ti0, ti1, ni0, ni1,
          hb0, hb1, tb0, tb1, nb0, nb1,
          gsem0, gsem1, osem0, osem1):
        hi_v = [hi0, hi1]
        ti_v = [ti0, ti1]
        ni_v = [ni0, ni1]
        hb = [hb0, hb1]
        tb = [tb0, tb1]
        nb = [nb0, nb1]
        gsem = [gsem0, gsem1]
        osem = [osem0, osem1]
        cid = lax.axis_index("c")
        sid = lax.axis_index("s")
        wid = cid * NS + sid
        base = wid * epw

        def outs(off, b):
            return [(hb[b], h_out), (tb[b], t_out), (nb[b], nt_out)]

        def stage(gq, q, b):
            def idx_body(j, carry2):
                sl = pl.ds(q * c + j * 16, 16)
                dl = pl.ds(j * 16, 16)
                hi_v[b][dl] = hib[sl]
                ti_v[b][dl] = tib[sl]
                ni_v[b][dl] = nib[sl]
                return carry2
            lax.fori_loop(0, c // 16, idx_body, 0)

            @pl.when(gq >= 2)
            def _():
                off_old = base
                for buf, dst in outs(off_old, b):
                    pltpu.make_async_copy(
                        buf, dst.at[pl.ds(off_old, c)], osem[b]).wait()

            pltpu.async_copy(x2_hbm.at[hi_v[b]], hb[b], gsem[b])
            pltpu.async_copy(x2_hbm.at[ti_v[b]], tb[b], gsem[b])
            pltpu.async_copy(x2_hbm.at[ni_v[b]], nb[b], gsem[b])

        def consume(off, b):
            pltpu.make_async_copy(x2_hbm.at[hi_v[b]], hb[b], gsem[b]).wait()
            pltpu.make_async_copy(x2_hbm.at[ti_v[b]], tb[b], gsem[b]).wait()
            pltpu.make_async_copy(x2_hbm.at[ni_v[b]], nb[b], gsem[b]).wait()
            for buf, dst in outs(off, b):
                pltpu.async_copy(buf, dst.at[pl.ds(off, c)], osem[b])

        for s in range(nsup):
            sup = base + s * _SUP * c
            pltpu.sync_copy(head_hbm.at[pl.ds(sup, _SUP * c)], hib)
            pltpu.sync_copy(tail_hbm.at[pl.ds(sup, _SUP * c)], tib)
            pltpu.sync_copy(nt_hbm.at[pl.ds(sup, _SUP * c)], nib)
            stage(s * _SUP, 0, 0)

            def pair_body(g2, carry):
                for b in range(2):
                    q = g2 * 2 + b

                    @pl.when(q + 1 < _SUP)
                    def _():
                        stage(s * _SUP + q + 1, q + 1, 1 - b)

                    @pl.when(q < _SUP)
                    def _():
                        consume(sup + q * c, b)
                return carry
            lax.fori_loop(0, -(-_SUP // 2), pair_body, 0)

        for b in range(2):
            for buf, dst in outs(base, b):
                pltpu.make_async_copy(
                    buf, dst.at[pl.ds(base, c)], osem[b]).wait()

    return k


def kernel(edge_index, edge_type, entity_embedding, entity_embedding_bias,
           basis1, comp1, root1, bias1, basis2, comp2, root2, bias2,
           rel_embedding):
    n, d = entity_embedding.shape
    nb, _, h = basis1.shape
    e = edge_type.shape[0]
    r_used = rel_embedding.shape[0]

    src = edge_index[0]
    dst = edge_index[1]
    nt_idx = jnp.roll(dst, 1)

    m1 = _build_m(comp1, basis1, root1, r_used)
    m2 = _build_m(comp2, basis2, root2, r_used)

    z1, rt1 = _pre_stage(entity_embedding, entity_embedding_bias, m1, r_used)
    z1 = z1.reshape(r_used * n, h + 16)

    edge_l1 = _make_edge_pass(n, e, r_used, h + 16)
    (aggp1,) = edge_l1(z1, src, dst, edge_type)

    z2, rt2 = _mid_stage(aggp1, rt1, bias1.reshape(1, h), m2, r_used)
    z2 = z2.reshape(r_used * n, d)

    edge_l2 = _make_edge_pass(n, e, r_used, d)
    (aggp2,) = edge_l2(z2, src, dst, edge_type)

    degp = aggp1[:, :, h:]
    x2 = _post_stage(aggp2, degp, rt2, bias2.reshape(1, d))

    decode = _make_decode(n, e, d)
    harr, tarr, ntarr = decode(x2, src, dst, nt_idx)

    pred_logits, loss, auc = _dec_reduce_stage(harr, tarr, ntarr,
                                               rel_embedding, edge_type)
    return (pred_logits, loss, auc)
